# baseline (speedup 1.0000x reference)
import jax
import jax.numpy as jnp
from jax import lax
from jax.experimental import pallas as pl
from jax.experimental.pallas import tpu as pltpu
from jax.experimental.pallas import tpu_sc as plsc

_D = 64
_NC = 2
_NS = 16
_NW = _NC * _NS
_BT_PER_W = 4
_S = 200
_NBLK = _S * 2
_SBB = 8


def _body(xT_hbm, tr_hbm, out_hbm,
          ib, ib2, r0, r1, t0, t1,
          isem, gs0, gs1, os0, os1):
    wid = lax.axis_index("s") * _NC + lax.axis_index("c")
    rows = (r0, r1)
    ts = (t0, t1)
    gsems = (gs0, gs1)
    osems = (os0, os1)
    bt0 = wid * _BT_PER_W
    col0 = bt0 * 128
    iota16 = lax.iota(jnp.int32, 16)
    bvs = [iota16 + g * 16 for g in range(16)]
    blvs = [iota16 + (g % 8) * 16 for g in range(16)]
    c16 = jnp.full((16,), 16, jnp.int32)

    def stage_idx(m):
        pltpu.async_copy(
            xT_hbm.at[pl.ds(m * 4, 4), pl.ds(col0, 512)],
            ib.at[lax.rem(m, 2)], isem)

    def wait_and_shift(m):
        pltpu.make_async_copy(
            xT_hbm.at[pl.ds(0, 4), pl.ds(0, 512)], ib.at[0], isem).wait()
        mp = lax.rem(m, 2)

        def sh(ls, carry):
            def sh2(c, carry2):
                v = ib[mp, ls, pl.ds(c * 16, 16)]
                ib2[mp, ls, pl.ds(c * 16, 16)] = lax.shift_right_logical(v, 1)
                return carry2
            return lax.fori_loop(0, 32, sh2, carry)
        lax.fori_loop(0, 4, sh, 0)

    def locate(k):
        mp = lax.rem(k // _SBB, 2)
        ls = lax.rem(k // 2, 4)
        po = lax.rem(k, 2) * 256
        return mp, ls, po

    def start_gather(k, p):
        mp, ls, po = locate(k)
        for h in range(2):
            pltpu.async_copy(
                tr_hbm.at[ib2.at[mp, ls, pl.ds(po + h * 128, 128)]],
                rows[p].at[pl.ds(h * 128, 128), :], gsems[p])

    def wait_gather(p):
        for h in range(2):
            pltpu.make_async_copy(
                tr_hbm.at[ib2.at[0, 0, pl.ds(0, 128)]],
                rows[p].at[pl.ds(h * 128, 128), :], gsems[p]).wait()

    def start_out(k, p):
        s = k // 2
        btp = bt0 + lax.rem(k, 2) * 2
        pltpu.async_copy(ts[p], out_hbm.at[s, :, pl.ds(btp, 2)], osems[p])

    def wait_out(p):
        pltpu.make_async_copy(
            ts[p], out_hbm.at[0, :, pl.ds(0, 2)], osems[p]).wait()

    def transpose(k, p):
        mp, ls, po = locate(k)
        r = rows[p]
        t = ts[p]
        pars = []
        for g in range(16):
            iv = ib[mp, ls, pl.ds(po + g * 16, 16)]
            pars.append(lax.shift_left(lax.bitwise_and(iv, 1), 6))

        def diag(jd, carry):
            rot_j = lax.rem(iota16 + jd, c16)
            for dq in range(4):
                idx_d = rot_j + dq * 16
                idx_dt = lax.shift_right_logical(idx_d, 3)
                idx_ds = lax.bitwise_and(idx_d, 7)
                for g in range(16):
                    v = plsc.load_gather(r, [bvs[g], idx_d + pars[g]])
                    plsc.store_scatter(
                        t, [idx_dt, jnp.full((16,), g // 8, jnp.int32),
                            idx_ds, blvs[g]], v)
            return carry

        lax.fori_loop(0, 16, diag, 0)

    stage_idx(0)
    stage_idx(1)
    wait_and_shift(0)
    start_gather(0, 0)

    def loop(j, carry):
        for i in range(2):
            k = 2 * j + i
            p = i
            wait_gather(p)
            if i == 0:
                @pl.when((k > 0) & (lax.rem(k, _SBB) == 0)
                         & (k + _SBB < _NBLK))
                def _():
                    stage_idx(k // _SBB + 1)

                @pl.when(k + 1 < _NBLK)
                def _():
                    start_gather(k + 1, 1)
            else:
                @pl.when(k + 1 < _NBLK)
                def _():
                    @pl.when(lax.rem(k + 1, _SBB) == 0)
                    def _():
                        wait_and_shift((k + 1) // _SBB)
                    start_gather(k + 1, 0)

            @pl.when(k >= 2)
            def _():
                wait_out(p)

            transpose(k, p)
            start_out(k, p)
        return carry

    lax.fori_loop(0, _NBLK // 2, loop, 0)
    wait_out(0)
    wait_out(1)


def kernel(x, table):
    b, s = x.shape
    xT = x.T.astype(jnp.int32)
    tr = table.reshape(table.shape[0] // 2, 128)
    mesh = plsc.VectorSubcoreMesh(core_axis_name="c", subcore_axis_name="s")
    out5 = pl.kernel(
        _body,
        out_type=jax.ShapeDtypeStruct((_S, 8, 128, 8, 128), jnp.float32),
        mesh=mesh,
        scratch_types=(
            [pltpu.VMEM((2, 4, 512), jnp.int32)] * 2
            + [pltpu.VMEM((256, 128), jnp.float32)] * 2
            + [pltpu.VMEM((8, 2, 8, 128), jnp.float32)] * 2
            + [pltpu.SemaphoreType.DMA] * 5
        ),
        compiler_params=pltpu.CompilerParams(
            use_tc_tiling_on_sc=True, needs_layout_passes=False),
    )(xT, tr)
    return out5.transpose(2, 4, 0, 1, 3).reshape(b, s, _D)


# speedup vs baseline: 1.4329x; 1.4329x over previous
import jax
import jax.numpy as jnp
from jax import lax
from jax.experimental import pallas as pl
from jax.experimental.pallas import tpu as pltpu
from jax.experimental.pallas import tpu_sc as plsc

_D = 64
_NC = 2
_NS = 16
_NW = _NC * _NS
_BT_PER_W = 4
_S = 200
_NBLK = _S * 2
_SBB = 8


def _body(xT_hbm, tr_hbm, out_hbm,
          ib, ib2, r0, r1, t0, t1,
          isem, gs0, gs1, os0, os1):
    wid = lax.axis_index("s") * _NC + lax.axis_index("c")
    rows = (r0, r1)
    ts = (t0, t1)
    gsems = (gs0, gs1)
    osems = (os0, os1)
    bt0 = wid * _BT_PER_W
    col0 = bt0 * 128
    iota16 = lax.iota(jnp.int32, 16)
    bvs = [iota16 + g * 16 for g in range(16)]
    blvs = [iota16 + (g % 8) * 16 for g in range(16)]
    c16 = jnp.full((16,), 16, jnp.int32)

    def stage_idx(m):
        pltpu.async_copy(
            xT_hbm.at[pl.ds(m * 4, 4), pl.ds(col0, 512)],
            ib.at[lax.rem(m, 2)], isem)

    def wait_and_shift(m):
        pltpu.make_async_copy(
            xT_hbm.at[pl.ds(0, 4), pl.ds(0, 512)], ib.at[0], isem).wait()
        mp = lax.rem(m, 2)

        def sh(ls, carry):
            def sh2(c, carry2):
                v = ib[mp, ls, pl.ds(c * 16, 16)]
                ib2[mp, ls, pl.ds(c * 16, 16)] = lax.shift_right_logical(v, 1)
                return carry2
            return lax.fori_loop(0, 32, sh2, carry)
        lax.fori_loop(0, 4, sh, 0)

    def locate(k):
        mp = lax.rem(k // _SBB, 2)
        ls = lax.rem(k // 2, 4)
        po = lax.rem(k, 2) * 256
        return mp, ls, po

    def start_gather(k, p):
        mp, ls, po = locate(k)
        for h in range(2):
            pltpu.async_copy(
                tr_hbm.at[ib2.at[mp, ls, pl.ds(po + h * 128, 128)]],
                rows[p].at[pl.ds(h * 128, 128), :], gsems[p])

    def wait_gather(p):
        for h in range(2):
            pltpu.make_async_copy(
                tr_hbm.at[ib2.at[0, 0, pl.ds(0, 128)]],
                rows[p].at[pl.ds(h * 128, 128), :], gsems[p]).wait()

    def start_out(k, p):
        s = k // 2
        btp = bt0 + lax.rem(k, 2) * 2
        pltpu.async_copy(ts[p], out_hbm.at[s, :, pl.ds(btp, 2)], osems[p])

    def wait_out(p):
        pltpu.make_async_copy(
            ts[p], out_hbm.at[0, :, pl.ds(0, 2)], osems[p]).wait()

    def transpose(k, p):
        mp, ls, po = locate(k)
        r = rows[p]
        t = ts[p]
        pars = []
        for g in range(16):
            iv = ib[mp, ls, pl.ds(po + g * 16, 16)]
            pars.append(lax.shift_left(lax.bitwise_and(iv, 1), 6))

        def diag(jd, carry):
            rot_j = lax.rem(iota16 + jd, c16)
            for dq in range(4):
                idx_d = rot_j + dq * 16
                idx_dt = lax.shift_right_logical(idx_d, 3)
                idx_ds = lax.bitwise_and(idx_d, 7)
                vs = [plsc.load_gather(r, [bvs[g], idx_d + pars[g]])
                      for g in range(16)]
                for g in range(16):
                    plsc.store_scatter(
                        t, [idx_dt, jnp.full((16,), g // 8, jnp.int32),
                            idx_ds, blvs[g]], vs[g])
            return carry

        lax.fori_loop(0, 16, diag, 0)

    stage_idx(0)
    stage_idx(1)
    wait_and_shift(0)
    start_gather(0, 0)

    def loop(j, carry):
        for i in range(2):
            k = 2 * j + i
            p = i
            wait_gather(p)
            if i == 0:
                @pl.when((k > 0) & (lax.rem(k, _SBB) == 0)
                         & (k + _SBB < _NBLK))
                def _():
                    stage_idx(k // _SBB + 1)

                @pl.when(k + 1 < _NBLK)
                def _():
                    start_gather(k + 1, 1)
            else:
                @pl.when(k + 1 < _NBLK)
                def _():
                    @pl.when(lax.rem(k + 1, _SBB) == 0)
                    def _():
                        wait_and_shift((k + 1) // _SBB)
                    start_gather(k + 1, 0)

            @pl.when(k >= 2)
            def _():
                wait_out(p)

            transpose(k, p)
            start_out(k, p)
        return carry

    lax.fori_loop(0, _NBLK // 2, loop, 0)
    wait_out(0)
    wait_out(1)


def kernel(x, table):
    b, s = x.shape
    xT = x.T.astype(jnp.int32)
    tr = table.reshape(table.shape[0] // 2, 128)
    mesh = plsc.VectorSubcoreMesh(core_axis_name="c", subcore_axis_name="s")
    out5 = pl.kernel(
        _body,
        out_type=jax.ShapeDtypeStruct((_S, 8, 128, 8, 128), jnp.float32),
        mesh=mesh,
        scratch_types=(
            [pltpu.VMEM((2, 4, 512), jnp.int32)] * 2
            + [pltpu.VMEM((256, 128), jnp.float32)] * 2
            + [pltpu.VMEM((8, 2, 8, 128), jnp.float32)] * 2
            + [pltpu.SemaphoreType.DMA] * 5
        ),
        compiler_params=pltpu.CompilerParams(
            use_tc_tiling_on_sc=True, needs_layout_passes=False),
    )(xT, tr)
    return out5.transpose(2, 4, 0, 1, 3).reshape(b, s, _D)


# speedup vs baseline: 1.4354x; 1.0018x over previous
import jax
import jax.numpy as jnp
from jax import lax
from jax.experimental import pallas as pl
from jax.experimental.pallas import tpu as pltpu
from jax.experimental.pallas import tpu_sc as plsc

_D = 64
_NC = 2
_NS = 16
_NW = _NC * _NS
_BT_PER_W = 4
_S = 200
_NBLK = _S * 2
_SBB = 8


def _body(xT_hbm, tr_hbm, out_hbm,
          ib, ib2, r0, r1, t0, t1,
          isem, gs0, gs1, os0, os1):
    wid = lax.axis_index("s") * _NC + lax.axis_index("c")
    rows = (r0, r1)
    ts = (t0, t1)
    gsems = (gs0, gs1)
    osems = (os0, os1)
    bt0 = wid * _BT_PER_W
    col0 = bt0 * 128
    iota16 = lax.iota(jnp.int32, 16)
    bvs = [iota16 + g * 16 for g in range(16)]
    blvs = [iota16 + (g % 8) * 16 for g in range(16)]
    btlv = [jnp.full((16,), g // 8, jnp.int32) for g in range(16)]
    c16 = jnp.full((16,), 16, jnp.int32)

    def stage_idx(m):
        pltpu.async_copy(
            xT_hbm.at[pl.ds(m * 4, 4), pl.ds(col0, 512)],
            ib.at[lax.rem(m, 2)], isem)

    def wait_and_shift(m):
        pltpu.make_async_copy(
            xT_hbm.at[pl.ds(0, 4), pl.ds(0, 512)], ib.at[0], isem).wait()
        mp = lax.rem(m, 2)

        def sh(ls, carry):
            def sh2(c, carry2):
                v = ib[mp, ls, pl.ds(c * 16, 16)]
                ib2[mp, ls, pl.ds(c * 16, 16)] = lax.shift_right_logical(v, 1)
                return carry2
            return lax.fori_loop(0, 32, sh2, carry)
        lax.fori_loop(0, 4, sh, 0)

    def locate(k):
        mp = lax.rem(k // _SBB, 2)
        ls = lax.rem(k // 2, 4)
        po = lax.rem(k, 2) * 256
        return mp, ls, po

    def start_gather(k, p):
        mp, ls, po = locate(k)
        for h in range(2):
            pltpu.async_copy(
                tr_hbm.at[ib2.at[mp, ls, pl.ds(po + h * 128, 128)]],
                rows[p].at[pl.ds(h * 128, 128), :], gsems[p])

    def wait_gather(p):
        for h in range(2):
            pltpu.make_async_copy(
                tr_hbm.at[ib2.at[0, 0, pl.ds(0, 128)]],
                rows[p].at[pl.ds(h * 128, 128), :], gsems[p]).wait()

    def start_out(k, p):
        s = k // 2
        btp = bt0 + lax.rem(k, 2) * 2
        pltpu.async_copy(ts[p], out_hbm.at[s, :, pl.ds(btp, 2)], osems[p])

    def wait_out(p):
        pltpu.make_async_copy(
            ts[p], out_hbm.at[0, :, pl.ds(0, 2)], osems[p]).wait()

    def transpose(k, p):
        mp, ls, po = locate(k)
        r = rows[p]
        t = ts[p]
        pars = []
        for g in range(16):
            iv = ib[mp, ls, pl.ds(po + g * 16, 16)]
            pars.append(lax.shift_left(lax.bitwise_and(iv, 1), 6))

        def diag(jd, carry):
            rot_j = lax.rem(iota16 + jd, c16)
            for dq2 in range(2):
                idx = []
                vs = []
                for dq in (2 * dq2, 2 * dq2 + 1):
                    idx_d = rot_j + dq * 16
                    idx.append((lax.shift_right_logical(idx_d, 3),
                                lax.bitwise_and(idx_d, 7)))
                    vs += [plsc.load_gather(r, [bvs[g], idx_d + pars[g]])
                           for g in range(16)]
                for q in range(2):
                    idx_dt, idx_ds = idx[q]
                    for g in range(16):
                        plsc.store_scatter(
                            t, [idx_dt, btlv[g], idx_ds, blvs[g]],
                            vs[q * 16 + g])
            return carry

        lax.fori_loop(0, 16, diag, 0)

    stage_idx(0)
    stage_idx(1)
    wait_and_shift(0)
    start_gather(0, 0)

    def loop(j, carry):
        for i in range(2):
            k = 2 * j + i
            p = i
            wait_gather(p)
            if i == 0:
                @pl.when((k > 0) & (lax.rem(k, _SBB) == 0)
                         & (k + _SBB < _NBLK))
                def _():
                    stage_idx(k // _SBB + 1)

                @pl.when(k + 1 < _NBLK)
                def _():
                    start_gather(k + 1, 1)
            else:
                @pl.when(k + 1 < _NBLK)
                def _():
                    @pl.when(lax.rem(k + 1, _SBB) == 0)
                    def _():
                        wait_and_shift((k + 1) // _SBB)
                    start_gather(k + 1, 0)

            @pl.when(k >= 2)
            def _():
                wait_out(p)

            transpose(k, p)
            start_out(k, p)
        return carry

    lax.fori_loop(0, _NBLK // 2, loop, 0)
    wait_out(0)
    wait_out(1)


def kernel(x, table):
    b, s = x.shape
    xT = x.T.astype(jnp.int32)
    tr = table.reshape(table.shape[0] // 2, 128)
    mesh = plsc.VectorSubcoreMesh(core_axis_name="c", subcore_axis_name="s")
    out5 = pl.kernel(
        _body,
        out_type=jax.ShapeDtypeStruct((_S, 8, 128, 8, 128), jnp.float32),
        mesh=mesh,
        scratch_types=(
            [pltpu.VMEM((2, 4, 512), jnp.int32)] * 2
            + [pltpu.VMEM((256, 128), jnp.float32)] * 2
            + [pltpu.VMEM((8, 2, 8, 128), jnp.float32)] * 2
            + [pltpu.SemaphoreType.DMA] * 5
        ),
        compiler_params=pltpu.CompilerParams(
            use_tc_tiling_on_sc=True, needs_layout_passes=False),
    )(xT, tr)
    return out5.transpose(2, 4, 0, 1, 3).reshape(b, s, _D)


# speedup vs baseline: 1.5366x; 1.0705x over previous
import jax
import jax.numpy as jnp
from jax import lax
from jax.experimental import pallas as pl
from jax.experimental.pallas import tpu as pltpu
from jax.experimental.pallas import tpu_sc as plsc

_D = 64
_NC = 2
_NS = 16
_NW = _NC * _NS
_BT_PER_W = 4
_S = 200
_NBLK = _S * _BT_PER_W
_SBB = 16


def _body(xT_hbm, tr_hbm, out_hbm,
          ib, ib2, r0, r1, r2, r3, t0, t1,
          isem, gs0, gs1, gs2, gs3, os0, os1):
    wid = lax.axis_index("s") * _NC + lax.axis_index("c")
    rows = (r0, r1, r2, r3)
    ts = (t0, t1)
    gsems = (gs0, gs1, gs2, gs3)
    osems = (os0, os1)
    bt0 = wid * _BT_PER_W
    col0 = bt0 * 128
    iota16 = lax.iota(jnp.int32, 16)
    bvs = [iota16 + g * 16 for g in range(8)]
    c16 = jnp.full((16,), 16, jnp.int32)

    def stage_idx(m):
        pltpu.async_copy(
            xT_hbm.at[pl.ds(m * 4, 4), pl.ds(col0, 512)],
            ib.at[lax.rem(m, 2)], isem)

    def wait_and_shift(m):
        pltpu.make_async_copy(
            xT_hbm.at[pl.ds(0, 4), pl.ds(0, 512)], ib.at[0], isem).wait()
        mp = lax.rem(m, 2)

        def sh(ls, carry):
            def sh2(c, carry2):
                v = ib[mp, ls, pl.ds(c * 16, 16)]
                ib2[mp, ls, pl.ds(c * 16, 16)] = lax.shift_right_logical(v, 1)
                return carry2
            return lax.fori_loop(0, 32, sh2, carry)
        lax.fori_loop(0, 4, sh, 0)

    def locate(k):
        mp = lax.rem(k // _SBB, 2)
        ls = lax.rem(k // _BT_PER_W, 4)
        po = lax.rem(k, _BT_PER_W) * 128
        return mp, ls, po

    def start_gather(k, q):
        mp, ls, po = locate(k)
        pltpu.async_copy(
            tr_hbm.at[ib2.at[mp, ls, pl.ds(po, 128)]], rows[q], gsems[q])

    def wait_gather(q):
        pltpu.make_async_copy(
            tr_hbm.at[ib2.at[0, 0, pl.ds(0, 128)]], rows[q], gsems[q]).wait()

    def start_out(k, p):
        s = k // _BT_PER_W
        bt = bt0 + lax.rem(k, _BT_PER_W)
        pltpu.async_copy(ts[p], out_hbm.at[s, :, bt], osems[p])

    def wait_out(p):
        pltpu.make_async_copy(ts[p], out_hbm.at[0, :, 0], osems[p]).wait()

    def transpose(k, q, p):
        mp, ls, po = locate(k)
        r = rows[q]
        t = ts[p]
        pars = []
        for g in range(8):
            iv = ib[mp, ls, pl.ds(po + g * 16, 16)]
            pars.append(lax.shift_left(lax.bitwise_and(iv, 1), 6))

        def diag(jd, carry):
            rot_j = lax.rem(iota16 + jd, c16)
            for dq2 in range(2):
                idx = []
                vs = []
                for dq in (2 * dq2, 2 * dq2 + 1):
                    idx_d = rot_j + dq * 16
                    idx.append((lax.shift_right_logical(idx_d, 3),
                                lax.bitwise_and(idx_d, 7)))
                    vs += [plsc.load_gather(r, [bvs[g], idx_d + pars[g]])
                           for g in range(8)]
                for u in range(2):
                    idx_dt, idx_ds = idx[u]
                    for g in range(8):
                        plsc.store_scatter(
                            t, [idx_dt, idx_ds, bvs[g]], vs[u * 8 + g])
            return carry

        lax.fori_loop(0, 16, diag, 0)

    stage_idx(0)
    stage_idx(1)
    wait_and_shift(0)
    start_gather(0, 0)
    start_gather(1, 1)
    start_gather(2, 2)

    def loop(j, carry):
        for i in range(4):
            k = 4 * j + i
            q = i
            p = i % 2
            wait_gather(q)

            @pl.when(k >= 2)
            def _():
                wait_out(p)

            transpose(k, q, p)
            start_out(k, p)

            if i == 0:
                @pl.when((k > 0) & (lax.rem(k, _SBB) == 0)
                         & (k + _SBB < _NBLK))
                def _():
                    stage_idx(k // _SBB + 1)

            @pl.when(k + 3 < _NBLK)
            def _():
                if i == 1:
                    @pl.when(lax.rem(k + 3, _SBB) == 0)
                    def _():
                        wait_and_shift((k + 3) // _SBB)
                start_gather(k + 3, (i + 3) % 4)
        return carry

    lax.fori_loop(0, _NBLK // 4, loop, 0)
    wait_out(0)
    wait_out(1)


def kernel(x, table):
    b, s = x.shape
    xT = x.T.astype(jnp.int32)
    tr = table.reshape(table.shape[0] // 2, 128)
    mesh = plsc.VectorSubcoreMesh(core_axis_name="c", subcore_axis_name="s")
    out5 = pl.kernel(
        _body,
        out_type=jax.ShapeDtypeStruct((_S, 8, 128, 8, 128), jnp.float32),
        mesh=mesh,
        scratch_types=(
            [pltpu.VMEM((2, 4, 512), jnp.int32)] * 2
            + [pltpu.VMEM((128, 128), jnp.float32)] * 4
            + [pltpu.VMEM((8, 8, 128), jnp.float32)] * 2
            + [pltpu.SemaphoreType.DMA] * 7
        ),
        compiler_params=pltpu.CompilerParams(
            use_tc_tiling_on_sc=True, needs_layout_passes=False),
    )(xT, tr)
    return out5.transpose(2, 4, 0, 1, 3).reshape(b, s, _D)


# speedup vs baseline: 1.5430x; 1.0042x over previous
import jax
import jax.numpy as jnp
from jax import lax
from jax.experimental import pallas as pl
from jax.experimental.pallas import tpu as pltpu
from jax.experimental.pallas import tpu_sc as plsc

_D = 64
_NC = 2
_NS = 16
_NW = _NC * _NS
_BT_PER_W = 4
_S = 200
_NBLK = _S * _BT_PER_W
_SBB = 16


def _body(xT_hbm, tr_hbm, out_hbm,
          ib, ib2, r0, r1, r2, r3, t0, t1,
          isem, gs0, gs1, gs2, gs3, os0, os1):
    wid = lax.axis_index("s") * _NC + lax.axis_index("c")
    rows = (r0, r1, r2, r3)
    ts = (t0, t1)
    gsems = (gs0, gs1, gs2, gs3)
    osems = (os0, os1)
    bt0 = wid * _BT_PER_W
    col0 = bt0 * 128
    iota16 = lax.iota(jnp.int32, 16)
    bvs = [iota16 + g * 16 for g in range(8)]
    c16 = jnp.full((16,), 16, jnp.int32)

    def stage_idx(m):
        pltpu.async_copy(
            xT_hbm.at[pl.ds(m * 4, 4), pl.ds(col0, 512)],
            ib.at[lax.rem(m, 2)], isem)

    def wait_and_shift(m):
        pltpu.make_async_copy(
            xT_hbm.at[pl.ds(0, 4), pl.ds(0, 512)], ib.at[0], isem).wait()
        mp = lax.rem(m, 2)

        def sh(ls, carry):
            def sh2(c, carry2):
                v = ib[mp, ls, pl.ds(c * 16, 16)]
                ib2[mp, ls, pl.ds(c * 16, 16)] = lax.shift_right_logical(v, 1)
                return carry2
            return lax.fori_loop(0, 32, sh2, carry)
        lax.fori_loop(0, 4, sh, 0)

    def locate(k):
        mp = lax.rem(k // _SBB, 2)
        ls = lax.rem(k // _BT_PER_W, 4)
        po = lax.rem(k, _BT_PER_W) * 128
        return mp, ls, po

    def start_gather(k, q):
        mp, ls, po = locate(k)
        pltpu.async_copy(
            tr_hbm.at[ib2.at[mp, ls, pl.ds(po, 128)]], rows[q], gsems[q])

    def wait_gather(q):
        pltpu.make_async_copy(
            tr_hbm.at[ib2.at[0, 0, pl.ds(0, 128)]], rows[q], gsems[q]).wait()

    def start_out(k, p):
        s = k // _BT_PER_W
        bt = bt0 + lax.rem(k, _BT_PER_W)
        pltpu.async_copy(ts[p], out_hbm.at[s, :, bt], osems[p])

    def wait_out(p):
        pltpu.make_async_copy(ts[p], out_hbm.at[0, :, 0], osems[p]).wait()

    def transpose(k, q, p):
        mp, ls, po = locate(k)
        r = rows[q]
        t = ts[p]
        pars = []
        for g in range(8):
            iv = ib[mp, ls, pl.ds(po + g * 16, 16)]
            pars.append(lax.shift_left(lax.bitwise_and(iv, 1), 6))

        def diag(jd, carry):
            rot_j = lax.rem(iota16 + jd, c16)
            for dq2 in range(2):
                idx = []
                vs = []
                for dq in (2 * dq2, 2 * dq2 + 1):
                    idx_d = rot_j + dq * 16
                    idx.append((lax.shift_right_logical(idx_d, 3),
                                lax.bitwise_and(idx_d, 7)))
                    vs += [plsc.load_gather(r, [bvs[g], idx_d + pars[g]])
                           for g in range(8)]
                for u in range(2):
                    idx_dt, idx_ds = idx[u]
                    for g in range(8):
                        plsc.store_scatter(
                            t, [idx_dt, idx_ds, bvs[g]], vs[u * 8 + g])
            return carry

        lax.fori_loop(0, 16, diag, 0)

    stage_idx(0)
    stage_idx(1)
    wait_and_shift(0)
    start_gather(0, 0)
    start_gather(1, 1)
    start_gather(2, 2)

    def loop(j, carry):
        for i in range(4):
            k = 4 * j + i
            q = i
            p = i % 2
            wait_gather(q)

            if i == 0:
                @pl.when((k > 0) & (lax.rem(k, _SBB) == 0)
                         & (k + _SBB < _NBLK))
                def _():
                    stage_idx(k // _SBB + 1)

            @pl.when(k + 3 < _NBLK)
            def _():
                if i == 1:
                    @pl.when(lax.rem(k + 3, _SBB) == 0)
                    def _():
                        wait_and_shift((k + 3) // _SBB)
                start_gather(k + 3, (i + 3) % 4)

            @pl.when(k >= 2)
            def _():
                wait_out(p)

            transpose(k, q, p)
            start_out(k, p)
        return carry

    lax.fori_loop(0, _NBLK // 4, loop, 0)
    wait_out(0)
    wait_out(1)


def kernel(x, table):
    b, s = x.shape
    xT = x.T.astype(jnp.int32)
    tr = table.reshape(table.shape[0] // 2, 128)
    mesh = plsc.VectorSubcoreMesh(core_axis_name="c", subcore_axis_name="s")
    out5 = pl.kernel(
        _body,
        out_type=jax.ShapeDtypeStruct((_S, 8, 128, 8, 128), jnp.float32),
        mesh=mesh,
        scratch_types=(
            [pltpu.VMEM((2, 4, 512), jnp.int32)] * 2
            + [pltpu.VMEM((128, 128), jnp.float32)] * 4
            + [pltpu.VMEM((8, 8, 128), jnp.float32)] * 2
            + [pltpu.SemaphoreType.DMA] * 7
        ),
        compiler_params=pltpu.CompilerParams(
            use_tc_tiling_on_sc=True, needs_layout_passes=False),
    )(xT, tr)
    return out5.transpose(2, 4, 0, 1, 3).reshape(b, s, _D)


# speedup vs baseline: 1.5451x; 1.0013x over previous
import jax
import jax.numpy as jnp
from jax import lax
from jax.experimental import pallas as pl
from jax.experimental.pallas import tpu as pltpu
from jax.experimental.pallas import tpu_sc as plsc

_D = 64
_NC = 2
_NS = 16
_NW = _NC * _NS
_BT_PER_W = 4
_S = 200
_NBLK = _S * _BT_PER_W
_SBB = 16


def _body(xT_hbm, tr_hbm, out_hbm,
          ib, ib2, r0, r1, r2, r3, t0, t1,
          isem, gs0, gs1, gs2, gs3, os0, os1):
    wid = lax.axis_index("s") * _NC + lax.axis_index("c")
    rows = (r0, r1, r2, r3)
    ts = (t0, t1)
    gsems = (gs0, gs1, gs2, gs3)
    osems = (os0, os1)
    bt0 = wid * _BT_PER_W
    col0 = bt0 * 128
    iota16 = lax.iota(jnp.int32, 16)
    bvs = [iota16 + g * 16 for g in range(8)]
    c16 = jnp.full((16,), 16, jnp.int32)

    def stage_idx(m):
        pltpu.async_copy(
            xT_hbm.at[pl.ds(m * 4, 4), pl.ds(col0, 512)],
            ib.at[lax.rem(m, 2)], isem)

    def wait_and_shift(m):
        pltpu.make_async_copy(
            xT_hbm.at[pl.ds(0, 4), pl.ds(0, 512)], ib.at[0], isem).wait()
        mp = lax.rem(m, 2)

        def sh(ls, carry):
            def sh2(c, carry2):
                v = ib[mp, ls, pl.ds(c * 16, 16)]
                ib2[mp, ls, pl.ds(c * 16, 16)] = lax.shift_right_logical(v, 1)
                return carry2
            return lax.fori_loop(0, 32, sh2, carry)
        lax.fori_loop(0, 4, sh, 0)

    def locate(k):
        mp = lax.rem(k // _SBB, 2)
        ls = lax.rem(k // _BT_PER_W, 4)
        po = lax.rem(k, _BT_PER_W) * 128
        return mp, ls, po

    def start_gather(k, q):
        mp, ls, po = locate(k)
        pltpu.async_copy(
            tr_hbm.at[ib2.at[mp, ls, pl.ds(po, 128)]], rows[q], gsems[q])

    def wait_gather(q):
        pltpu.make_async_copy(
            tr_hbm.at[ib2.at[0, 0, pl.ds(0, 128)]], rows[q], gsems[q]).wait()

    def start_out(k, p):
        s = k // _BT_PER_W
        bt = bt0 + lax.rem(k, _BT_PER_W)
        pltpu.async_copy(ts[p], out_hbm.at[s, :, bt], osems[p])

    def wait_out(p):
        pltpu.make_async_copy(ts[p], out_hbm.at[0, :, 0], osems[p]).wait()

    def transpose(k, q, p):
        mp, ls, po = locate(k)
        r = rows[q]
        t = ts[p]
        pars = []
        for g in range(8):
            iv = ib[mp, ls, pl.ds(po + g * 16, 16)]
            pars.append(lax.shift_left(lax.bitwise_and(iv, 1), 6))

        def diag(jd, carry):
            rot_j = lax.rem(iota16 + jd, c16)
            for dq2 in range(2):
                idx = []
                vs = []
                for dq in (2 * dq2, 2 * dq2 + 1):
                    idx_d = rot_j + dq * 16
                    idx.append((lax.shift_right_logical(idx_d, 3),
                                lax.bitwise_and(idx_d, 7)))
                    vs += [plsc.load_gather(r, [bvs[g], idx_d + pars[g]])
                           for g in range(8)]
                for u in range(2):
                    idx_dt, idx_ds = idx[u]
                    for g in range(8):
                        plsc.store_scatter(
                            t, [idx_dt, idx_ds, bvs[g]], vs[u * 8 + g])
            return carry

        lax.fori_loop(0, 16, diag, 0)

    stage_idx(0)
    wait_and_shift(0)
    stage_idx(1)
    start_gather(0, 0)
    start_gather(1, 1)
    start_gather(2, 2)

    def loop(j, carry):
        for i in range(4):
            k = 4 * j + i
            q = i
            p = i % 2
            wait_gather(q)

            if i == 0:
                @pl.when((k > 0) & (lax.rem(k, _SBB) == 0)
                         & (k + _SBB < _NBLK))
                def _():
                    stage_idx(k // _SBB + 1)

            @pl.when(k + 3 < _NBLK)
            def _():
                if i == 1:
                    @pl.when(lax.rem(k + 3, _SBB) == 0)
                    def _():
                        wait_and_shift((k + 3) // _SBB)
                start_gather(k + 3, (i + 3) % 4)

            @pl.when(k >= 2)
            def _():
                wait_out(p)

            transpose(k, q, p)
            start_out(k, p)
        return carry

    lax.fori_loop(0, _NBLK // 4, loop, 0)
    wait_out(0)
    wait_out(1)


def kernel(x, table):
    b, s = x.shape
    xT = x.T.astype(jnp.int32)
    tr = table.reshape(table.shape[0] // 2, 128)
    mesh = plsc.VectorSubcoreMesh(core_axis_name="c", subcore_axis_name="s")
    out5 = pl.kernel(
        _body,
        out_type=jax.ShapeDtypeStruct((_S, 8, 128, 8, 128), jnp.float32),
        mesh=mesh,
        scratch_types=(
            [pltpu.VMEM((2, 4, 512), jnp.int32)] * 2
            + [pltpu.VMEM((128, 128), jnp.float32)] * 4
            + [pltpu.VMEM((8, 8, 128), jnp.float32)] * 2
            + [pltpu.SemaphoreType.DMA] * 7
        ),
        compiler_params=pltpu.CompilerParams(
            use_tc_tiling_on_sc=True, needs_layout_passes=False),
    )(xT, tr)
    return out5.transpose(2, 4, 0, 1, 3).reshape(b, s, _D)
